# Initial kernel scaffold; baseline (speedup 1.0000x reference)
import functools

import jax
import jax.numpy as jnp
from jax.experimental import pallas as pl
from jax.experimental.pallas import tpu as pltpu

_KS = (1, 3, 5, 7, 9, 11, 13)
_NK = len(_KS)
_PAD = 8
_NEG = -jnp.inf


def _means_body(x_ref, o_ref):
    s = jnp.sum(x_ref[...], axis=(1, 2))
    o_ref[0, :] = s


def _router_body(content_ref, w1_ref, b1_ref, w2_ref, b2_ref,
                 alpha_ref, nlev_ref):
    content = content_ref[...]
    hidden = jnp.maximum(
        jax.lax.dot_general(content, w1_ref[...],
                            (((1,), (1,)), ((), ())),
                            preferred_element_type=jnp.float32)
        + b1_ref[0, :][None, :], 0.0)
    glob = jax.lax.dot_general(hidden, w2_ref[...],
                               (((1,), (1,)), ((), ())),
                               preferred_element_type=jnp.float32) \
        + b2_ref[0, :][None, :]
    e = jnp.maximum(glob, 0.0)
    q_s = jnp.clip(jnp.floor(e), 0.0, float(_NK - 2))
    w_big = e - q_s
    w_small = (q_s + 1.0) - e
    for i in range(_NK):
        fi = float(i)
        alpha = jnp.where(q_s == fi, w_small, 0.0) \
            + jnp.where(q_s == fi - 1.0, w_big, 0.0)
        alpha_ref[i, :, :] = alpha
    nlev_ref[...] = (q_s + 1.0).astype(jnp.int32)


def _pool_body(alpha_ref, nlev_ref, x_ref, o_ref, a_ref, b_ref, *, h, w, c):
    bc = pl.program_id(0)
    bi = bc // c
    ci = bc % c

    hp = h + 2 * _PAD
    wp = w + 2 * _PAD

    a_ref[...] = jnp.full((1, hp, wp), _NEG, jnp.float32)
    b_ref[...] = jnp.full((1, hp, wp), _NEG, jnp.float32)
    x = x_ref[0]
    a_ref[0, _PAD:_PAD + h, _PAD:_PAD + w] = x

    a0 = alpha_ref[0, bi, ci]
    o_ref[0] = (1.0 + a0) * x

    nlev = nlev_ref[bi, ci]
    for i in range(1, _NK):
        @pl.when(i <= nlev)
        def _():
            v = a_ref[0]
            b_ref[0, :, 1:wp - 1] = jnp.maximum(
                jnp.maximum(v[:, 0:wp - 2], v[:, 1:wp - 1]), v[:, 2:wp])
            u = b_ref[0]
            a_ref[0, 1:hp - 1, :] = jnp.maximum(
                jnp.maximum(u[0:hp - 2, :], u[1:hp - 1, :]), u[2:hp, :])
            ai = alpha_ref[i, bi, ci]
            o_ref[0] = o_ref[0] + ai * a_ref[0, _PAD:_PAD + h, _PAD:_PAD + w]


def kernel(x, W1, b1, W2, b2):
    b, c, h, w = x.shape
    cr = W1.shape[0]
    bc = b * c
    xf = x.reshape(bc, h, w)

    cb = 16
    assert bc % cb == 0
    sums = pl.pallas_call(
        _means_body,
        grid=(bc // cb,),
        in_specs=[pl.BlockSpec((cb, h, w), lambda i: (i, 0, 0))],
        out_specs=pl.BlockSpec((1, cb), lambda i: (i, 0)),
        out_shape=jax.ShapeDtypeStruct((bc // cb, cb), jnp.float32),
    )(xf)
    content = sums.reshape(b, c) * (1.0 / (h * w))

    alpha, nlev = pl.pallas_call(
        _router_body,
        out_shape=(jax.ShapeDtypeStruct((_NK, b, c), jnp.float32),
                   jax.ShapeDtypeStruct((b, c), jnp.int32)),
    )(content, W1, b1.reshape(1, cr), W2, b2.reshape(1, c))

    hp, wp = h + 2 * _PAD, w + 2 * _PAD
    out = pl.pallas_call(
        functools.partial(_pool_body, h=h, w=w, c=c),
        grid=(bc,),
        in_specs=[
            pl.BlockSpec(memory_space=pltpu.SMEM),
            pl.BlockSpec(memory_space=pltpu.SMEM),
            pl.BlockSpec((1, h, w), lambda i: (i, 0, 0)),
        ],
        out_specs=pl.BlockSpec((1, h, w), lambda i: (i, 0, 0)),
        out_shape=jax.ShapeDtypeStruct((bc, h, w), jnp.float32),
        scratch_shapes=[pltpu.VMEM((1, hp, wp), jnp.float32),
                        pltpu.VMEM((1, hp, wp), jnp.float32)],
    )(alpha, nlev, xf)
    return out.reshape(b, c, h, w)


# speedup vs baseline: 4.6130x; 4.6130x over previous
import functools

import jax
import jax.numpy as jnp
from jax.experimental import pallas as pl
from jax.experimental.pallas import tpu as pltpu

_KS = (1, 3, 5, 7, 9, 11, 13)
_NK = len(_KS)
_PAD = 8
_NEG = -jnp.inf


def _means_body(x_ref, o_ref):
    s = jnp.sum(x_ref[...], axis=(1, 2))
    o_ref[0, 0, :] = s


def _router_body(content_ref, w1_ref, b1_ref, w2_ref, b2_ref,
                 alpha_ref, nlev_ref):
    content = content_ref[...]
    hidden = jnp.maximum(
        jax.lax.dot_general(content, w1_ref[...],
                            (((1,), (1,)), ((), ())),
                            preferred_element_type=jnp.float32)
        + b1_ref[0, :][None, :], 0.0)
    glob = jax.lax.dot_general(hidden, w2_ref[...],
                               (((1,), (1,)), ((), ())),
                               preferred_element_type=jnp.float32) \
        + b2_ref[0, :][None, :]
    e = jnp.maximum(glob, 0.0)
    q_s = jnp.clip(jnp.floor(e), 0.0, float(_NK - 2))
    w_big = e - q_s
    w_small = (q_s + 1.0) - e
    for i in range(_NK):
        fi = float(i)
        alpha = jnp.where(q_s == fi, w_small, 0.0) \
            + jnp.where(q_s == fi - 1.0, w_big, 0.0)
        alpha_ref[i, :, :] = alpha
    nlev_ref[...] = (q_s + 1.0).astype(jnp.int32)


def _pool_body(alpha_ref, nlev_ref, x_ref, o_ref, a_ref, b_ref, *, h, w, c):
    bc = pl.program_id(0)
    bi = bc // c
    ci = bc % c

    hp = h + 2 * _PAD
    wp = w + 2 * _PAD

    a_ref[...] = jnp.full((1, hp, wp), _NEG, jnp.float32)
    b_ref[...] = jnp.full((1, hp, wp), _NEG, jnp.float32)
    x = x_ref[0]
    a_ref[0, _PAD:_PAD + h, _PAD:_PAD + w] = x

    a0 = alpha_ref[0, bi, ci]
    o_ref[0] = (1.0 + a0) * x

    nlev = nlev_ref[bi, ci]
    for i in range(1, _NK):
        @pl.when(i <= nlev)
        def _():
            v = a_ref[0]
            b_ref[0, :, 1:wp - 1] = jnp.maximum(
                jnp.maximum(v[:, 0:wp - 2], v[:, 1:wp - 1]), v[:, 2:wp])
            u = b_ref[0]
            a_ref[0, 1:hp - 1, :] = jnp.maximum(
                jnp.maximum(u[0:hp - 2, :], u[1:hp - 1, :]), u[2:hp, :])
            ai = alpha_ref[i, bi, ci]
            o_ref[0] = o_ref[0] + ai * a_ref[0, _PAD:_PAD + h, _PAD:_PAD + w]


def kernel(x, W1, b1, W2, b2):
    b, c, h, w = x.shape
    cr = W1.shape[0]
    bc = b * c
    xf = x.reshape(bc, h, w)

    cb = 16
    assert bc % cb == 0
    sums = pl.pallas_call(
        _means_body,
        grid=(bc // cb,),
        in_specs=[pl.BlockSpec((cb, h, w), lambda i: (i, 0, 0))],
        out_specs=pl.BlockSpec((1, 1, cb), lambda i: (i, 0, 0)),
        out_shape=jax.ShapeDtypeStruct((bc // cb, 1, cb), jnp.float32),
    )(xf)
    content = sums.reshape(b, c) * (1.0 / (h * w))

    alpha, nlev = pl.pallas_call(
        _router_body,
        out_shape=(jax.ShapeDtypeStruct((_NK, b, c), jnp.float32),
                   jax.ShapeDtypeStruct((b, c), jnp.int32)),
    )(content, W1, b1.reshape(1, cr), W2, b2.reshape(1, c))

    hp, wp = h + 2 * _PAD, w + 2 * _PAD
    out = pl.pallas_call(
        functools.partial(_pool_body, h=h, w=w, c=c),
        grid=(bc,),
        in_specs=[
            pl.BlockSpec(memory_space=pltpu.SMEM),
            pl.BlockSpec(memory_space=pltpu.SMEM),
            pl.BlockSpec((1, h, w), lambda i: (i, 0, 0)),
        ],
        out_specs=pl.BlockSpec((1, h, w), lambda i: (i, 0, 0)),
        out_shape=jax.ShapeDtypeStruct((bc, h, w), jnp.float32),
        scratch_shapes=[pltpu.VMEM((1, hp, wp), jnp.float32),
                        pltpu.VMEM((1, hp, wp), jnp.float32)],
    )(alpha, nlev, xf)
    return out.reshape(b, c, h, w)


# speedup vs baseline: 4.8742x; 1.0566x over previous
import functools

import jax
import jax.numpy as jnp
from jax.experimental import pallas as pl
from jax.experimental.pallas import tpu as pltpu

_KS = (1, 3, 5, 7, 9, 11, 13)
_NK = len(_KS)
_PAD = 8
_NEG = -jnp.inf


def _means_body(x_ref, o_ref):
    s = jnp.sum(x_ref[...], axis=(1, 2))
    o_ref[0, 0, :] = s


def _router_body(content_ref, w1_ref, b1_ref, w2_ref, b2_ref,
                 alpha_ref, nlev_ref):
    content = content_ref[...]
    hidden = jnp.maximum(
        jax.lax.dot_general(content, w1_ref[...],
                            (((1,), (1,)), ((), ())),
                            preferred_element_type=jnp.float32)
        + b1_ref[0, :][None, :], 0.0)
    glob = jax.lax.dot_general(hidden, w2_ref[...],
                               (((1,), (1,)), ((), ())),
                               preferred_element_type=jnp.float32) \
        + b2_ref[0, :][None, :]
    e = jnp.maximum(glob, 0.0)
    q_s = jnp.clip(jnp.floor(e), 0.0, float(_NK - 2))
    w_big = e - q_s
    w_small = (q_s + 1.0) - e
    for i in range(_NK):
        fi = float(i)
        alpha = jnp.where(q_s == fi, w_small, 0.0) \
            + jnp.where(q_s == fi - 1.0, w_big, 0.0)
        alpha_ref[i, :, :] = alpha
    nlev_ref[...] = (q_s + 1.0).astype(jnp.int32)


def _pool_body(alpha_ref, nlev_ref, x_ref, o_ref, a_ref, b_ref, *, h, w, c):
    g = pl.program_id(0)
    bi = g // c
    ci = g % c

    hp = h + 2 * _PAD
    wp = w + 2 * _PAD

    @pl.when(g == 0)
    def _():
        b_ref[...] = jnp.full((1, hp, wp), _NEG, jnp.float32)

    a_ref[0, 0:_PAD, :] = jnp.full((_PAD, wp), _NEG, jnp.float32)
    a_ref[0, _PAD + h:, :] = jnp.full((_PAD, wp), _NEG, jnp.float32)
    a_ref[0, :, 0:_PAD] = jnp.full((hp, _PAD), _NEG, jnp.float32)
    a_ref[0, :, _PAD + w:] = jnp.full((hp, _PAD), _NEG, jnp.float32)
    x = x_ref[0]
    a_ref[0, _PAD:_PAD + h, _PAD:_PAD + w] = x

    a0 = alpha_ref[0, bi, ci]
    o_ref[0] = (1.0 + a0) * x

    nlev = nlev_ref[bi, ci]
    for i in range(1, _NK):
        r = (_NK - 1) - i

        @pl.when(i <= nlev)
        def _():
            lo = _PAD - r - 1
            hi = _PAD + h + r + 1
            cl = _PAD - r
            ch = _PAD + w + r
            v = a_ref[0]
            b_ref[0, lo:hi, cl:ch] = jnp.maximum(
                jnp.maximum(v[lo:hi, cl - 1:ch - 1], v[lo:hi, cl:ch]),
                v[lo:hi, cl + 1:ch + 1])
            u = b_ref[0]
            a_ref[0, cl:_PAD + h + r, cl:ch] = jnp.maximum(
                jnp.maximum(u[lo:_PAD + h + r - 1, cl:ch],
                            u[cl:_PAD + h + r, cl:ch]),
                u[cl + 1:hi, cl:ch])

            @pl.when(i >= nlev - 1)
            def _():
                ai = alpha_ref[i, bi, ci]
                o_ref[0] = o_ref[0] + ai * a_ref[0, _PAD:_PAD + h,
                                                 _PAD:_PAD + w]


def kernel(x, W1, b1, W2, b2):
    b, c, h, w = x.shape
    cr = W1.shape[0]
    bc = b * c
    xf = x.reshape(bc, h, w)

    cb = 16
    assert bc % cb == 0
    sums = pl.pallas_call(
        _means_body,
        grid=(bc // cb,),
        in_specs=[pl.BlockSpec((cb, h, w), lambda i: (i, 0, 0))],
        out_specs=pl.BlockSpec((1, 1, cb), lambda i: (i, 0, 0)),
        out_shape=jax.ShapeDtypeStruct((bc // cb, 1, cb), jnp.float32),
    )(xf)
    content = sums.reshape(b, c) * (1.0 / (h * w))

    alpha, nlev = pl.pallas_call(
        _router_body,
        out_shape=(jax.ShapeDtypeStruct((_NK, b, c), jnp.float32),
                   jax.ShapeDtypeStruct((b, c), jnp.int32)),
    )(content, W1, b1.reshape(1, cr), W2, b2.reshape(1, c))

    hp, wp = h + 2 * _PAD, w + 2 * _PAD
    out = pl.pallas_call(
        functools.partial(_pool_body, h=h, w=w, c=c),
        grid=(bc,),
        in_specs=[
            pl.BlockSpec(memory_space=pltpu.SMEM),
            pl.BlockSpec(memory_space=pltpu.SMEM),
            pl.BlockSpec((1, h, w), lambda i: (i, 0, 0)),
        ],
        out_specs=pl.BlockSpec((1, h, w), lambda i: (i, 0, 0)),
        out_shape=jax.ShapeDtypeStruct((bc, h, w), jnp.float32),
        scratch_shapes=[pltpu.VMEM((1, hp, wp), jnp.float32),
                        pltpu.VMEM((1, hp, wp), jnp.float32)],
    )(alpha, nlev, xf)
    return out.reshape(b, c, h, w)
